# baseline (speedup 1.0000x reference)
import functools

import jax
import jax.numpy as jnp
from jax.experimental import pallas as pl
from jax.experimental.pallas import tpu as pltpu

_VMEM_LIMIT = 64 * 1024 * 1024

_DN_T = (((1,), (1,)), ((), ()))


def _fused_mha_kernel(x_ref, wq_ref, wk_ref, wv_ref, wo_ref, o_ref,
                      wq_s, wk_s, wv_s, wo_s,
                      *, num_heads, head_dim, scaling):
    f32 = jnp.float32
    cdt = jnp.bfloat16

    @pl.when(pl.program_id(0) == 0)
    def _():
        wq_s[...] = (wq_ref[...] * scaling).astype(cdt)
        wk_s[...] = wk_ref[...].astype(cdt)
        wv_s[...] = wv_ref[...].astype(cdt)
        wo_s[...] = wo_ref[...].astype(cdt)

    x = x_ref[...].astype(cdt)

    q = jax.lax.dot_general(x, wq_s[...], _DN_T,
                            preferred_element_type=f32).astype(cdt)
    k = jax.lax.dot_general(x, wk_s[...], _DN_T,
                            preferred_element_type=f32).astype(cdt)
    v = jax.lax.dot_general(x, wv_s[...], _DN_T,
                            preferred_element_type=f32).astype(cdt)

    t = x.shape[0]
    tq = 128
    for r in range(t // tq):
        rows = slice(r * tq, (r + 1) * tq)
        outs = []
        for h in range(num_heads):
            sl = slice(h * head_dim, (h + 1) * head_dim)
            s = jax.lax.dot_general(q[rows, sl], k[:, sl], _DN_T,
                                    preferred_element_type=f32)
            m = jnp.max(s, axis=-1, keepdims=True)
            p = jnp.exp(s - m)
            l = jnp.sum(p, axis=-1, keepdims=True)
            acc = jnp.dot(p.astype(cdt), v[:, sl],
                          preferred_element_type=f32)
            outs.append((acc * pl.reciprocal(l, approx=False)).astype(cdt))

        attn_r = jnp.concatenate(outs, axis=-1)
        out_r = jax.lax.dot_general(attn_r, wo_s[...], _DN_T,
                                    preferred_element_type=f32)
        o_ref[rows, :] = out_r.astype(cdt).astype(o_ref.dtype)


def kernel(hidden_states, wq, wk, wv, wo):
    B, T, E = hidden_states.shape
    num_heads = 16
    head_dim = E // num_heads
    scaling = head_dim ** (-0.5)
    orig_dtype = hidden_states.dtype
    cdt = jnp.bfloat16

    cost = pl.CostEstimate(
        flops=2 * B * T * E * E * 4 + 4 * B * num_heads * T * T * head_dim,
        transcendentals=B * num_heads * T * T,
        bytes_accessed=B * T * E * 8 + 4 * E * E * 4,
    )

    fused = functools.partial(
        _fused_mha_kernel, num_heads=num_heads, head_dim=head_dim,
        scaling=scaling)

    out = pl.pallas_call(
        fused,
        out_shape=jax.ShapeDtypeStruct((B, T, E), orig_dtype),
        grid_spec=pltpu.PrefetchScalarGridSpec(
            num_scalar_prefetch=0,
            grid=(B,),
            in_specs=[
                pl.BlockSpec((None, T, E), lambda b: (b, 0, 0)),
                pl.BlockSpec((E, E), lambda b: (0, 0)),
                pl.BlockSpec((E, E), lambda b: (0, 0)),
                pl.BlockSpec((E, E), lambda b: (0, 0)),
                pl.BlockSpec((E, E), lambda b: (0, 0)),
            ],
            out_specs=pl.BlockSpec((None, T, E), lambda b: (b, 0, 0)),
            scratch_shapes=[
                pltpu.VMEM((E, E), cdt),
                pltpu.VMEM((E, E), cdt),
                pltpu.VMEM((E, E), cdt),
                pltpu.VMEM((E, E), cdt),
            ],
        ),
        compiler_params=pltpu.CompilerParams(
            dimension_semantics=("arbitrary",),
            vmem_limit_bytes=_VMEM_LIMIT,
        ),
        cost_estimate=cost,
    )(hidden_states, wq, wk, wv, wo)
    return out


# speedup vs baseline: 1.2690x; 1.2690x over previous
import functools

import jax
import jax.numpy as jnp
from jax.experimental import pallas as pl
from jax.experimental.pallas import tpu as pltpu

_VMEM_LIMIT = 64 * 1024 * 1024

_DN_T = (((1,), (1,)), ((), ()))


def _fused_mha_kernel(x_ref, wq_ref, wk_ref, wv_ref, wo_ref, o_ref,
                      wq_s, wk_s, wv_s, wo_s,
                      *, num_heads, head_dim, scaling):
    f32 = jnp.float32
    cdt = jnp.bfloat16

    @pl.when(pl.program_id(0) == 0)
    def _():
        wq_s[...] = (wq_ref[...] * scaling).astype(cdt)
        wk_s[...] = wk_ref[...].astype(cdt)
        wv_s[...] = wv_ref[...].astype(cdt)
        wo_s[...] = wo_ref[...].astype(cdt)

    x = x_ref[...].astype(cdt)

    q = jax.lax.dot_general(x, wq_s[...], _DN_T,
                            preferred_element_type=f32).astype(cdt)
    k = jax.lax.dot_general(x, wk_s[...], _DN_T,
                            preferred_element_type=f32).astype(cdt)
    v = jax.lax.dot_general(x, wv_s[...], _DN_T,
                            preferred_element_type=f32).astype(cdt)

    t = x.shape[0]
    tq = 256
    for r in range(t // tq):
        rows = slice(r * tq, (r + 1) * tq)
        outs = []
        for h in range(num_heads):
            sl = slice(h * head_dim, (h + 1) * head_dim)
            s = jax.lax.dot_general(q[rows, sl], k[:, sl], _DN_T,
                                    preferred_element_type=f32)
            m = jnp.max(s, axis=-1, keepdims=True)
            p = jnp.exp(s - m)
            l = jnp.sum(p, axis=-1, keepdims=True)
            acc = jnp.dot(p.astype(cdt), v[:, sl],
                          preferred_element_type=f32)
            outs.append((acc * pl.reciprocal(l, approx=False)).astype(cdt))

        attn_r = jnp.concatenate(outs, axis=-1)
        out_r = jax.lax.dot_general(attn_r, wo_s[...], _DN_T,
                                    preferred_element_type=f32)
        o_ref[rows, :] = out_r.astype(cdt).astype(o_ref.dtype)


def kernel(hidden_states, wq, wk, wv, wo):
    B, T, E = hidden_states.shape
    num_heads = 16
    head_dim = E // num_heads
    scaling = head_dim ** (-0.5)
    orig_dtype = hidden_states.dtype
    cdt = jnp.bfloat16

    cost = pl.CostEstimate(
        flops=2 * B * T * E * E * 4 + 4 * B * num_heads * T * T * head_dim,
        transcendentals=B * num_heads * T * T,
        bytes_accessed=B * T * E * 8 + 4 * E * E * 4,
    )

    fused = functools.partial(
        _fused_mha_kernel, num_heads=num_heads, head_dim=head_dim,
        scaling=scaling)

    out = pl.pallas_call(
        fused,
        out_shape=jax.ShapeDtypeStruct((B, T, E), orig_dtype),
        grid_spec=pltpu.PrefetchScalarGridSpec(
            num_scalar_prefetch=0,
            grid=(B,),
            in_specs=[
                pl.BlockSpec((None, T, E), lambda b: (b, 0, 0)),
                pl.BlockSpec((E, E), lambda b: (0, 0)),
                pl.BlockSpec((E, E), lambda b: (0, 0)),
                pl.BlockSpec((E, E), lambda b: (0, 0)),
                pl.BlockSpec((E, E), lambda b: (0, 0)),
            ],
            out_specs=pl.BlockSpec((None, T, E), lambda b: (b, 0, 0)),
            scratch_shapes=[
                pltpu.VMEM((E, E), cdt),
                pltpu.VMEM((E, E), cdt),
                pltpu.VMEM((E, E), cdt),
                pltpu.VMEM((E, E), cdt),
            ],
        ),
        compiler_params=pltpu.CompilerParams(
            dimension_semantics=("arbitrary",),
            vmem_limit_bytes=_VMEM_LIMIT,
        ),
        cost_estimate=cost,
    )(hidden_states, wq, wk, wv, wo)
    return out


# speedup vs baseline: 2.0263x; 1.5968x over previous
import functools

import jax
import jax.numpy as jnp
from jax.experimental import pallas as pl
from jax.experimental.pallas import tpu as pltpu

_VMEM_LIMIT = 64 * 1024 * 1024

_DN_T = (((1,), (1,)), ((), ()))


def _fused_mha_kernel(x_ref, wq_ref, wk_ref, wv_ref, wo_ref, o_ref,
                      wq_s, wk_s, wv_s, wo_s,
                      *, num_heads, head_dim, scaling):
    f32 = jnp.float32
    cdt = jnp.bfloat16

    @pl.when(pl.program_id(0) == 0)
    def _():
        wq_s[...] = (wq_ref[...] * scaling).astype(cdt)
        wk_s[...] = wk_ref[...].astype(cdt)
        wv_s[...] = wv_ref[...].astype(cdt)
        wo_s[...] = wo_ref[...].astype(cdt)

    x = x_ref[...].astype(cdt)

    q = jax.lax.dot_general(x, wq_s[...], _DN_T,
                            preferred_element_type=f32).astype(cdt)
    k = jax.lax.dot_general(x, wk_s[...], _DN_T,
                            preferred_element_type=f32).astype(cdt)
    v = jax.lax.dot_general(x, wv_s[...], _DN_T,
                            preferred_element_type=f32).astype(cdt)

    outs = []
    for h in range(num_heads):
        sl = slice(h * head_dim, (h + 1) * head_dim)
        qh, kh, vh = q[:, sl], k[:, sl], v[:, sl]
        s = jax.lax.dot_general(qh, kh, _DN_T,
                                preferred_element_type=f32)
        m = jnp.max(s, axis=-1, keepdims=True)
        p = jnp.exp(s - m)
        l = jnp.sum(p, axis=-1, keepdims=True)
        acc = jnp.dot(p.astype(cdt), vh, preferred_element_type=f32)
        outs.append((acc * pl.reciprocal(l, approx=False)).astype(cdt))

    attn = jnp.concatenate(outs, axis=-1)

    out = jax.lax.dot_general(attn, wo_s[...], _DN_T,
                              preferred_element_type=f32)
    o_ref[...] = out.astype(cdt).astype(o_ref.dtype)


def kernel(hidden_states, wq, wk, wv, wo):
    B, T, E = hidden_states.shape
    num_heads = 16
    head_dim = E // num_heads
    scaling = head_dim ** (-0.5)
    orig_dtype = hidden_states.dtype
    cdt = jnp.bfloat16

    cost = pl.CostEstimate(
        flops=2 * B * T * E * E * 4 + 4 * B * num_heads * T * T * head_dim,
        transcendentals=B * num_heads * T * T,
        bytes_accessed=B * T * E * 8 + 4 * E * E * 4,
    )

    fused = functools.partial(
        _fused_mha_kernel, num_heads=num_heads, head_dim=head_dim,
        scaling=scaling)

    out = pl.pallas_call(
        fused,
        out_shape=jax.ShapeDtypeStruct((B, T, E), orig_dtype),
        grid_spec=pltpu.PrefetchScalarGridSpec(
            num_scalar_prefetch=0,
            grid=(B,),
            in_specs=[
                pl.BlockSpec((None, T, E), lambda b: (b, 0, 0)),
                pl.BlockSpec((E, E), lambda b: (0, 0)),
                pl.BlockSpec((E, E), lambda b: (0, 0)),
                pl.BlockSpec((E, E), lambda b: (0, 0)),
                pl.BlockSpec((E, E), lambda b: (0, 0)),
            ],
            out_specs=pl.BlockSpec((None, T, E), lambda b: (b, 0, 0)),
            scratch_shapes=[
                pltpu.VMEM((E, E), cdt),
                pltpu.VMEM((E, E), cdt),
                pltpu.VMEM((E, E), cdt),
                pltpu.VMEM((E, E), cdt),
            ],
        ),
        compiler_params=pltpu.CompilerParams(
            dimension_semantics=("arbitrary",),
            vmem_limit_bytes=_VMEM_LIMIT,
        ),
        cost_estimate=cost,
    )(hidden_states, wq, wk, wv, wo)
    return out
